# Initial kernel scaffold; baseline (speedup 1.0000x reference)
import jax
import jax.numpy as jnp
from jax.experimental import pallas as pl


def kernel(x_a, x_b, ea_w1, ea_b1, ea_g1, ea_be1, ea_w2, ea_b2, ea_g2, ea_be2, ea_dw1, ea_db1, ea_dw2, ea_db2, eb_w1, eb_b1, eb_g1, eb_be1, eb_w2, eb_b2, eb_g2, eb_be2, eb_dw1, eb_db1, eb_dw2, eb_db2, codebook):
    raise NotImplementedError("write your pallas kernel here")


# speedup vs baseline: 1.3894x; 1.3894x over previous
import functools

import jax
import jax.numpy as jnp
from jax import lax
from jax.experimental import pallas as pl
from jax.experimental.pallas import tpu as pltpu
from jax.experimental.pallas import tpu_sc as plsc

_F32 = jnp.float32


def _ln(x, g, b, eps=1e-5):
    m = jnp.mean(x, axis=-1, keepdims=True)
    v = jnp.mean((x - m) ** 2, axis=-1, keepdims=True)
    return (x - m) / jnp.sqrt(v + eps) * g + b


def _encode(x, w1, b1, g1, be1, w2, b2, g2, be2):
    h = jax.nn.relu(_ln(jnp.dot(x, w1, preferred_element_type=_F32) + b1, g1, be1))
    return _ln(jnp.dot(h, w2, preferred_element_type=_F32) + b2, g2, be2)


def _vq_search(z, cb_ref, e2_ref, bt, num_codes, kt):
    z2 = jnp.sum(z * z, axis=1, keepdims=True)
    cur_min = jnp.full((bt, 1), jnp.inf, dtype=_F32)
    cur_idx = jnp.zeros((bt, 1), dtype=jnp.int32)
    for t in range(num_codes // kt):
        cbt = cb_ref[pl.ds(t * kt, kt), :]
        ze = lax.dot_general(z, cbt, (((1,), (1,)), ((), ())),
                             preferred_element_type=_F32)
        d = z2 - 2.0 * ze + e2_ref[:, pl.ds(t * kt, kt)]
        m = jnp.min(d, axis=1, keepdims=True)
        it = lax.broadcasted_iota(jnp.int32, (bt, kt), 1)
        am = jnp.min(jnp.where(d == m, it, kt), axis=1, keepdims=True) + t * kt
        upd = m < cur_min
        cur_idx = jnp.where(upd, am, cur_idx)
        cur_min = jnp.where(upd, m, cur_min)
    return cur_idx


def _enc_vq_body(bt, num_codes, kt,
                 xa_ref, xb_ref,
                 aw1, ab1, ag1, abe1, aw2, ab2, ag2, abe2,
                 bw1, bb1, bg1, bbe1, bw2, bb2, bg2, bbe2,
                 cb_ref,
                 za_ref, zb_ref, ia_ref, ib_ref,
                 e2_ref):
    i = pl.program_id(0)

    @pl.when(i == 0)
    def _():
        cb = cb_ref[...]
        cd = cb.shape[1]
        ones = jnp.ones((1, cd), dtype=_F32)
        e2_ref[...] = lax.dot_general(ones, cb * cb, (((1,), (1,)), ((), ())),
                                      preferred_element_type=_F32)

    za = _encode(xa_ref[...], aw1[...], ab1[...], ag1[...], abe1[...],
                 aw2[...], ab2[...], ag2[...], abe2[...])
    zb = _encode(xb_ref[...], bw1[...], bb1[...], bg1[...], bbe1[...],
                 bw2[...], bb2[...], bg2[...], bbe2[...])
    za_ref[...] = za
    zb_ref[...] = zb
    ia_ref[...] = _vq_search(za, cb_ref, e2_ref, bt, num_codes, kt)
    ib_ref[...] = _vq_search(zb, cb_ref, e2_ref, bt, num_codes, kt)


def _dec_body(nsteps, inv_n,
              za_ref, qra_ref, zb_ref, qrb_ref,
              adw1, adb1, adw2, adb2,
              bdw1, bdb1, bdw2, bdb2,
              qsa_ref, qsb_ref, ra_ref, rb_ref, ca_ref, cbm_ref):
    i = pl.program_id(0)
    za = za_ref[...]
    zb = zb_ref[...]
    qra = qra_ref[...]
    qrb = qrb_ref[...]
    qsa = za + (qra - za)
    qsb = zb + (qrb - zb)
    qsa_ref[...] = qsa
    qsb_ref[...] = qsb
    ha = jax.nn.relu(jnp.dot(qsa, adw1[...], preferred_element_type=_F32) + adb1[...])
    ra_ref[...] = jnp.dot(ha, adw2[...], preferred_element_type=_F32) + adb2[...]
    hb = jax.nn.relu(jnp.dot(qsb, bdw1[...], preferred_element_type=_F32) + bdb1[...])
    rb_ref[...] = jnp.dot(hb, bdw2[...], preferred_element_type=_F32) + bdb2[...]

    da = za - qra
    db = zb - qrb
    pa = jnp.sum(da * da).reshape(1, 1)
    pb = jnp.sum(db * db).reshape(1, 1)

    @pl.when(i == 0)
    def _():
        ca_ref[...] = jnp.zeros((1, 1), _F32)
        cbm_ref[...] = jnp.zeros((1, 1), _F32)

    ca_ref[...] += pa
    cbm_ref[...] += pb

    @pl.when(i == nsteps - 1)
    def _():
        ca_ref[...] = ca_ref[...] * inv_n
        cbm_ref[...] = cbm_ref[...] * inv_n


def _make_sc_gather(num_codes, cd, b):
    nc, ns = 2, 16
    nw = nc * ns
    bpw = b // nw
    ch = 128 if bpw % 128 == 0 else bpw
    nch = bpw // ch
    mesh = plsc.VectorSubcoreMesh(core_axis_name="c", subcore_axis_name="s")

    @functools.partial(
        pl.kernel, mesh=mesh,
        out_type=[jax.ShapeDtypeStruct((b, cd), _F32),
                  jax.ShapeDtypeStruct((b, cd), _F32)],
        scratch_types=[pltpu.VMEM((ch,), jnp.int32),
                       pltpu.VMEM((ch, cd), _F32),
                       pltpu.SemaphoreType.DMA])
    def gather(cb_hbm, ia_hbm, ib_hbm, qa_hbm, qb_hbm, idx_v, rows_v, sem):
        wid = lax.axis_index("s") * nc + lax.axis_index("c")
        base = wid * bpw
        for src, dst in ((ia_hbm, qa_hbm), (ib_hbm, qb_hbm)):
            for c in range(nch):
                off = base + c * ch
                pltpu.sync_copy(src.at[pl.ds(off, ch)], idx_v)
                pltpu.async_copy(cb_hbm.at[idx_v], rows_v, sem).wait()
                pltpu.sync_copy(rows_v, dst.at[pl.ds(off, ch)])

    return gather


def kernel(x_a, x_b,
           ea_w1, ea_b1, ea_g1, ea_be1, ea_w2, ea_b2, ea_g2, ea_be2,
           ea_dw1, ea_db1, ea_dw2, ea_db2,
           eb_w1, eb_b1, eb_g1, eb_be1, eb_w2, eb_b2, eb_g2, eb_be2,
           eb_dw1, eb_db1, eb_dw2, eb_db2,
           codebook):
    b, dim_a = x_a.shape
    dim_b = x_b.shape[1]
    hid = ea_w1.shape[1]
    cd = ea_w2.shape[1]
    num_codes = codebook.shape[0]
    bt = 512 if b % 512 == 0 else b
    kt = 1024 if num_codes % 1024 == 0 else num_codes
    nsteps = b // bt

    row = lambda v: v.reshape(1, -1)
    full = lambda shape: pl.BlockSpec(shape, lambda i: (0,) * len(shape))
    batch = lambda shape: pl.BlockSpec(shape, lambda i: (i,) + (0,) * (len(shape) - 1))

    enc = pl.pallas_call(
        functools.partial(_enc_vq_body, bt, num_codes, kt),
        grid=(nsteps,),
        in_specs=[
            batch((bt, dim_a)), batch((bt, dim_b)),
            full((dim_a, hid)), full((1, hid)), full((1, hid)), full((1, hid)),
            full((hid, cd)), full((1, cd)), full((1, cd)), full((1, cd)),
            full((dim_b, hid)), full((1, hid)), full((1, hid)), full((1, hid)),
            full((hid, cd)), full((1, cd)), full((1, cd)), full((1, cd)),
            full((num_codes, cd)),
        ],
        out_specs=[batch((bt, cd)), batch((bt, cd)),
                   batch((bt, 1)), batch((bt, 1))],
        out_shape=[jax.ShapeDtypeStruct((b, cd), _F32),
                   jax.ShapeDtypeStruct((b, cd), _F32),
                   jax.ShapeDtypeStruct((b, 1), jnp.int32),
                   jax.ShapeDtypeStruct((b, 1), jnp.int32)],
        scratch_shapes=[pltpu.VMEM((1, num_codes), _F32)],
        compiler_params=pltpu.CompilerParams(
            dimension_semantics=("arbitrary",)),
    )
    z_a, z_b, ia2, ib2 = enc(
        x_a, x_b,
        ea_w1, row(ea_b1), row(ea_g1), row(ea_be1),
        ea_w2, row(ea_b2), row(ea_g2), row(ea_be2),
        eb_w1, row(eb_b1), row(eb_g1), row(eb_be1),
        eb_w2, row(eb_b2), row(eb_g2), row(eb_be2),
        codebook)
    idx_a = ia2.reshape(b)
    idx_b = ib2.reshape(b)

    q_raw_a, q_raw_b = _make_sc_gather(num_codes, cd, b)(codebook, idx_a, idx_b)

    dec = pl.pallas_call(
        functools.partial(_dec_body, nsteps, 1.0 / (b * cd)),
        grid=(nsteps,),
        in_specs=[
            batch((bt, cd)), batch((bt, cd)), batch((bt, cd)), batch((bt, cd)),
            full((cd, hid)), full((1, hid)), full((hid, dim_a)), full((1, dim_a)),
            full((cd, hid)), full((1, hid)), full((hid, dim_b)), full((1, dim_b)),
        ],
        out_specs=[batch((bt, cd)), batch((bt, cd)),
                   batch((bt, dim_a)), batch((bt, dim_b)),
                   full((1, 1)), full((1, 1))],
        out_shape=[jax.ShapeDtypeStruct((b, cd), _F32),
                   jax.ShapeDtypeStruct((b, cd), _F32),
                   jax.ShapeDtypeStruct((b, dim_a), _F32),
                   jax.ShapeDtypeStruct((b, dim_b), _F32),
                   jax.ShapeDtypeStruct((1, 1), _F32),
                   jax.ShapeDtypeStruct((1, 1), _F32)],
        compiler_params=pltpu.CompilerParams(
            dimension_semantics=("arbitrary",)),
    )
    q_a, q_b, recon_a, recon_b, ca, cb = dec(
        z_a, q_raw_a, z_b, q_raw_b,
        ea_dw1, row(ea_db1), ea_dw2, row(ea_db2),
        eb_dw1, row(eb_db1), eb_dw2, row(eb_db2))

    return (z_a, z_b, q_a, q_b, idx_a, idx_b, recon_a, recon_b,
            ca.reshape(()), cb.reshape(()))


# speedup vs baseline: 1.3912x; 1.0013x over previous
import functools

import jax
import jax.numpy as jnp
from jax import lax
from jax.experimental import pallas as pl
from jax.experimental.pallas import tpu as pltpu
from jax.experimental.pallas import tpu_sc as plsc

_F32 = jnp.float32
_BF16 = jnp.bfloat16


def _ln(x, g, b, eps=1e-5):
    m = jnp.mean(x, axis=-1, keepdims=True)
    v = jnp.mean((x - m) ** 2, axis=-1, keepdims=True)
    return (x - m) / jnp.sqrt(v + eps) * g + b


def _vq_search(z, cb_ref, e2_ref, bt, num_codes, kt):
    z2 = jnp.sum(z * z, axis=1, keepdims=True)
    cur_min = jnp.full((bt, 1), jnp.inf, dtype=_F32)
    cur_idx = jnp.zeros((bt, 1), dtype=jnp.int32)
    for t in range(num_codes // kt):
        cbt = cb_ref[pl.ds(t * kt, kt), :]
        ze = lax.dot_general(z, cbt, (((1,), (1,)), ((), ())),
                             preferred_element_type=_F32)
        d = z2 - 2.0 * ze + e2_ref[:, pl.ds(t * kt, kt)]
        m = jnp.min(d, axis=1, keepdims=True)
        it = lax.broadcasted_iota(jnp.int32, (bt, kt), 1)
        am = jnp.min(jnp.where(d == m, it, kt), axis=1, keepdims=True) + t * kt
        upd = m < cur_min
        cur_idx = jnp.where(upd, am, cur_idx)
        cur_min = jnp.where(upd, m, cur_min)
    return cur_idx


def _enc_vq_body(bt, num_codes, kt,
                 x_ref, w1, b1, g1, be1, w2, b2, g2, be2, cb_ref,
                 z_ref, i_ref, e2_ref):
    i = pl.program_id(0)

    @pl.when(i == 0)
    def _():
        cb = cb_ref[...]
        ones = jnp.ones((1, cb.shape[1]), dtype=_F32)
        e2_ref[...] = lax.dot_general(ones, cb * cb, (((1,), (1,)), ((), ())),
                                      preferred_element_type=_F32)

    h = jax.nn.relu(_ln(jnp.dot(x_ref[...], w1[...],
                                preferred_element_type=_F32) + b1[...],
                        g1[...], be1[...]))
    z = _ln(jnp.dot(h, w2[...], preferred_element_type=_F32) + b2[...],
            g2[...], be2[...])
    z_ref[...] = z
    i_ref[...] = _vq_search(z, cb_ref, e2_ref, bt, num_codes, kt)


def _dec_body(nsteps, inv_n,
              z_ref, qr_ref, dw1, db1, dw2, db2,
              qs_ref, r_ref, c_ref):
    i = pl.program_id(0)
    z = z_ref[...]
    qr = qr_ref[...]
    qs = z + (qr - z)
    qs_ref[...] = qs
    h = jax.nn.relu(jnp.dot(qs.astype(_BF16), dw1[...],
                            preferred_element_type=_F32) + db1[...])
    r_ref[...] = jnp.dot(h.astype(_BF16), dw2[...],
                         preferred_element_type=_F32) + db2[...]

    dz = z - qr
    p = jnp.sum(dz * dz).reshape(1, 1)

    @pl.when(i == 0)
    def _():
        c_ref[...] = jnp.zeros((1, 1), _F32)

    c_ref[...] += p

    @pl.when(i == nsteps - 1)
    def _():
        c_ref[...] = c_ref[...] * inv_n


def _make_sc_gather(cd, b):
    nc, ns = 2, 16
    nw = nc * ns
    bpw = b // nw
    ch = 128 if bpw % 128 == 0 else bpw
    nch = bpw // ch
    assert nch == 2, "pipelined gather below is written for 2 chunks/worker"
    mesh = plsc.VectorSubcoreMesh(core_axis_name="c", subcore_axis_name="s")

    @functools.partial(
        pl.kernel, mesh=mesh,
        out_type=jax.ShapeDtypeStruct((b, cd), _F32),
        scratch_types=[pltpu.VMEM((ch,), jnp.int32),
                       pltpu.VMEM((ch,), jnp.int32),
                       pltpu.VMEM((ch, cd), _F32),
                       pltpu.VMEM((ch, cd), _F32),
                       pltpu.SemaphoreType.DMA,
                       pltpu.SemaphoreType.DMA,
                       pltpu.SemaphoreType.DMA,
                       pltpu.SemaphoreType.DMA])
    def gather(cb_hbm, idx_hbm, q_hbm,
               i0, i1, r0, r1, sg0, sg1, ss0, ss1):
        wid = lax.axis_index("s") * nc + lax.axis_index("c")
        base = wid * bpw
        pltpu.sync_copy(idx_hbm.at[pl.ds(base, ch)], i0)
        pltpu.sync_copy(idx_hbm.at[pl.ds(base + ch, ch)], i1)
        g0 = pltpu.async_copy(cb_hbm.at[i0], r0, sg0)
        g1 = pltpu.async_copy(cb_hbm.at[i1], r1, sg1)
        g0.wait()
        s0 = pltpu.async_copy(r0, q_hbm.at[pl.ds(base, ch)], ss0)
        g1.wait()
        s1 = pltpu.async_copy(r1, q_hbm.at[pl.ds(base + ch, ch)], ss1)
        s0.wait()
        s1.wait()

    return gather


def _make_enc(b, bt, dim, hid, cd, num_codes, kt):
    nsteps = b // bt
    full = lambda shape: pl.BlockSpec(shape, lambda i: (0,) * len(shape))
    batch = lambda shape: pl.BlockSpec(shape, lambda i: (i,) + (0,) * (len(shape) - 1))
    return pl.pallas_call(
        functools.partial(_enc_vq_body, bt, num_codes, kt),
        grid=(nsteps,),
        in_specs=[
            batch((bt, dim)),
            full((dim, hid)), full((1, hid)), full((1, hid)), full((1, hid)),
            full((hid, cd)), full((1, cd)), full((1, cd)), full((1, cd)),
            full((num_codes, cd)),
        ],
        out_specs=[batch((bt, cd)), batch((bt, 1))],
        out_shape=[jax.ShapeDtypeStruct((b, cd), _F32),
                   jax.ShapeDtypeStruct((b, 1), jnp.int32)],
        scratch_shapes=[pltpu.VMEM((1, num_codes), _F32)],
        compiler_params=pltpu.CompilerParams(
            dimension_semantics=("arbitrary",)),
    )


def _make_dec(b, bt, dim, hid, cd):
    nsteps = b // bt
    full = lambda shape: pl.BlockSpec(shape, lambda i: (0,) * len(shape))
    batch = lambda shape: pl.BlockSpec(shape, lambda i: (i,) + (0,) * (len(shape) - 1))
    return pl.pallas_call(
        functools.partial(_dec_body, nsteps, 1.0 / (b * cd)),
        grid=(nsteps,),
        in_specs=[
            batch((bt, cd)), batch((bt, cd)),
            full((cd, hid)), full((1, hid)), full((hid, dim)), full((1, dim)),
        ],
        out_specs=[batch((bt, cd)), batch((bt, dim)), full((1, 1))],
        out_shape=[jax.ShapeDtypeStruct((b, cd), _F32),
                   jax.ShapeDtypeStruct((b, dim), _F32),
                   jax.ShapeDtypeStruct((1, 1), _F32)],
        compiler_params=pltpu.CompilerParams(
            dimension_semantics=("arbitrary",)),
    )


def kernel(x_a, x_b,
           ea_w1, ea_b1, ea_g1, ea_be1, ea_w2, ea_b2, ea_g2, ea_be2,
           ea_dw1, ea_db1, ea_dw2, ea_db2,
           eb_w1, eb_b1, eb_g1, eb_be1, eb_w2, eb_b2, eb_g2, eb_be2,
           eb_dw1, eb_db1, eb_dw2, eb_db2,
           codebook):
    b, dim_a = x_a.shape
    dim_b = x_b.shape[1]
    hid = ea_w1.shape[1]
    cd = ea_w2.shape[1]
    num_codes = codebook.shape[0]
    bt = 512 if b % 512 == 0 else b
    kt = 1024 if num_codes % 1024 == 0 else num_codes

    row = lambda v: v.reshape(1, -1)

    enc_a = _make_enc(b, bt, dim_a, hid, cd, num_codes, kt)
    enc_b = _make_enc(b, bt, dim_b, hid, cd, num_codes, kt)
    z_a, ia2 = enc_a(x_a, ea_w1, row(ea_b1), row(ea_g1), row(ea_be1),
                     ea_w2, row(ea_b2), row(ea_g2), row(ea_be2), codebook)
    z_b, ib2 = enc_b(x_b, eb_w1, row(eb_b1), row(eb_g1), row(eb_be1),
                     eb_w2, row(eb_b2), row(eb_g2), row(eb_be2), codebook)
    idx_a = ia2.reshape(b)
    idx_b = ib2.reshape(b)

    gather = _make_sc_gather(cd, b)
    q_raw_a = gather(codebook, idx_a)
    q_raw_b = gather(codebook, idx_b)

    dec_a = _make_dec(b, bt, dim_a, hid, cd)
    dec_b = _make_dec(b, bt, dim_b, hid, cd)
    q_a, recon_a, ca = dec_a(z_a, q_raw_a,
                             ea_dw1.astype(_BF16), row(ea_db1),
                             ea_dw2.astype(_BF16), row(ea_db2))
    q_b, recon_b, cb = dec_b(z_b, q_raw_b,
                             eb_dw1.astype(_BF16), row(eb_db1),
                             eb_dw2.astype(_BF16), row(eb_db2))

    return (z_a, z_b, q_a, q_b, idx_a, idx_b, recon_a, recon_b,
            ca.reshape(()), cb.reshape(()))


# speedup vs baseline: 1.6320x; 1.1731x over previous
import functools

import jax
import jax.numpy as jnp
from jax import lax
from jax.experimental import pallas as pl
from jax.experimental.pallas import tpu as pltpu
from jax.experimental.pallas import tpu_sc as plsc

_F32 = jnp.float32
_BF16 = jnp.bfloat16


def _ln(x, g, b, eps=1e-5):
    m = jnp.mean(x, axis=-1, keepdims=True)
    v = jnp.mean((x - m) ** 2, axis=-1, keepdims=True)
    return (x - m) / jnp.sqrt(v + eps) * g + b


def _vq_search(z, cb_ref, e2_ref, it_ref, bt, num_codes, kt):
    hz2 = 0.5 * jnp.sum(z * z, axis=1, keepdims=True)
    cur_idx = None
    for t in range(num_codes // kt):
        cbt = cb_ref[:, pl.ds(t * kt, kt)]
        ze = lax.dot_general(z, cbt, (((1,), (0,)), ((), ())),
                             preferred_element_type=_F32)
        d = hz2 - ze + e2_ref[:, pl.ds(t * kt, kt)]
        m = jnp.min(d, axis=1, keepdims=True)
        am = jnp.min(jnp.where(d == m, it_ref[:, pl.ds(t * kt, kt)], float(kt)),
                     axis=1, keepdims=True) + float(t * kt)
        if cur_idx is None:
            cur_min, cur_idx = m, am
        else:
            upd = m < cur_min
            cur_idx = jnp.where(upd, am, cur_idx)
            cur_min = jnp.where(upd, m, cur_min)
    return cur_idx.astype(jnp.int32)


def _enc_vq_body(bt, num_codes, kt,
                 x_ref, w1, b1, g1, be1, w2, b2, g2, be2, cb_ref,
                 z_ref, i_ref, e2_ref, it_ref):
    i = pl.program_id(0)

    @pl.when(i == 0)
    def _():
        cb = cb_ref[...]
        ones = jnp.ones((1, cb.shape[0]), dtype=_F32)
        e2_ref[...] = 0.5 * lax.dot_general(
            ones, cb * cb, (((1,), (0,)), ((), ())),
            preferred_element_type=_F32)
        it_ref[...] = lax.broadcasted_iota(
            jnp.int32, (1, num_codes), 1).astype(_F32)

    h = jax.nn.relu(_ln(jnp.dot(x_ref[...], w1[...],
                                preferred_element_type=_F32) + b1[...],
                        g1[...], be1[...]))
    z = _ln(jnp.dot(h, w2[...], preferred_element_type=_F32) + b2[...],
            g2[...], be2[...])
    z_ref[...] = z
    i_ref[...] = _vq_search(z, cb_ref, e2_ref, it_ref, bt, num_codes, kt)


def _dec_body(nsteps, inv_n,
              z_ref, qr_ref, dw1, db1, dw2, db2,
              qs_ref, r_ref, c_ref):
    i = pl.program_id(0)
    z = z_ref[...]
    qr = qr_ref[...]
    qs = z + (qr - z)
    qs_ref[...] = qs
    h = jax.nn.relu(jnp.dot(qs.astype(_BF16), dw1[...],
                            preferred_element_type=_F32) + db1[...])
    r_ref[...] = jnp.dot(h.astype(_BF16), dw2[...],
                         preferred_element_type=_F32) + db2[...]

    dz = z - qr
    p = jnp.sum(dz * dz).reshape(1, 1)

    @pl.when(i == 0)
    def _():
        c_ref[...] = jnp.zeros((1, 1), _F32)

    c_ref[...] += p

    @pl.when(i == nsteps - 1)
    def _():
        c_ref[...] = c_ref[...] * inv_n


def _make_sc_gather(cd, b):
    nc, ns = 2, 16
    nw = nc * ns
    bpw = b // nw
    ch = 128 if bpw % 128 == 0 else bpw
    nch = bpw // ch
    assert nch == 2, "pipelined gather below is written for 2 chunks/worker"
    mesh = plsc.VectorSubcoreMesh(core_axis_name="c", subcore_axis_name="s")

    @functools.partial(
        pl.kernel, mesh=mesh,
        out_type=jax.ShapeDtypeStruct((b, cd), _F32),
        scratch_types=[pltpu.VMEM((ch,), jnp.int32),
                       pltpu.VMEM((ch,), jnp.int32),
                       pltpu.VMEM((ch, cd), _F32),
                       pltpu.VMEM((ch, cd), _F32),
                       pltpu.SemaphoreType.DMA,
                       pltpu.SemaphoreType.DMA,
                       pltpu.SemaphoreType.DMA,
                       pltpu.SemaphoreType.DMA])
    def gather(cb_hbm, idx_hbm, q_hbm,
               i0, i1, r0, r1, sg0, sg1, ss0, ss1):
        wid = lax.axis_index("s") * nc + lax.axis_index("c")
        base = wid * bpw
        pltpu.sync_copy(idx_hbm.at[pl.ds(base, ch)], i0)
        pltpu.sync_copy(idx_hbm.at[pl.ds(base + ch, ch)], i1)
        g0 = pltpu.async_copy(cb_hbm.at[i0], r0, sg0)
        g1 = pltpu.async_copy(cb_hbm.at[i1], r1, sg1)
        g0.wait()
        s0 = pltpu.async_copy(r0, q_hbm.at[pl.ds(base, ch)], ss0)
        g1.wait()
        s1 = pltpu.async_copy(r1, q_hbm.at[pl.ds(base + ch, ch)], ss1)
        s0.wait()
        s1.wait()

    return gather


def _make_enc(b, bt, dim, hid, cd, num_codes, kt):
    nsteps = b // bt
    full = lambda shape: pl.BlockSpec(shape, lambda i: (0,) * len(shape))
    batch = lambda shape: pl.BlockSpec(shape, lambda i: (i,) + (0,) * (len(shape) - 1))
    return pl.pallas_call(
        functools.partial(_enc_vq_body, bt, num_codes, kt),
        grid=(nsteps,),
        in_specs=[
            batch((bt, dim)),
            full((dim, hid)), full((1, hid)), full((1, hid)), full((1, hid)),
            full((hid, cd)), full((1, cd)), full((1, cd)), full((1, cd)),
            full((cd, num_codes)),
        ],
        out_specs=[batch((bt, cd)), batch((bt, 1))],
        out_shape=[jax.ShapeDtypeStruct((b, cd), _F32),
                   jax.ShapeDtypeStruct((b, 1), jnp.int32)],
        scratch_shapes=[pltpu.VMEM((1, num_codes), _F32),
                        pltpu.VMEM((1, num_codes), _F32)],
        compiler_params=pltpu.CompilerParams(
            dimension_semantics=("arbitrary",)),
    )


def _make_dec(b, bt, dim, hid, cd):
    nsteps = b // bt
    full = lambda shape: pl.BlockSpec(shape, lambda i: (0,) * len(shape))
    batch = lambda shape: pl.BlockSpec(shape, lambda i: (i,) + (0,) * (len(shape) - 1))
    return pl.pallas_call(
        functools.partial(_dec_body, nsteps, 1.0 / (b * cd)),
        grid=(nsteps,),
        in_specs=[
            batch((bt, cd)), batch((bt, cd)),
            full((cd, hid)), full((1, hid)), full((hid, dim)), full((1, dim)),
        ],
        out_specs=[batch((bt, cd)), batch((bt, dim)), full((1, 1))],
        out_shape=[jax.ShapeDtypeStruct((b, cd), _F32),
                   jax.ShapeDtypeStruct((b, dim), _F32),
                   jax.ShapeDtypeStruct((1, 1), _F32)],
        compiler_params=pltpu.CompilerParams(
            dimension_semantics=("arbitrary",)),
    )


def kernel(x_a, x_b,
           ea_w1, ea_b1, ea_g1, ea_be1, ea_w2, ea_b2, ea_g2, ea_be2,
           ea_dw1, ea_db1, ea_dw2, ea_db2,
           eb_w1, eb_b1, eb_g1, eb_be1, eb_w2, eb_b2, eb_g2, eb_be2,
           eb_dw1, eb_db1, eb_dw2, eb_db2,
           codebook):
    b, dim_a = x_a.shape
    dim_b = x_b.shape[1]
    hid = ea_w1.shape[1]
    cd = ea_w2.shape[1]
    num_codes = codebook.shape[0]
    bt = 512 if b % 512 == 0 else b
    kt = num_codes

    row = lambda v: v.reshape(1, -1)

    cb_t = codebook.T
    enc_a = _make_enc(b, bt, dim_a, hid, cd, num_codes, kt)
    enc_b = _make_enc(b, bt, dim_b, hid, cd, num_codes, kt)
    z_a, ia2 = enc_a(x_a, ea_w1, row(ea_b1), row(ea_g1), row(ea_be1),
                     ea_w2, row(ea_b2), row(ea_g2), row(ea_be2), cb_t)
    z_b, ib2 = enc_b(x_b, eb_w1, row(eb_b1), row(eb_g1), row(eb_be1),
                     eb_w2, row(eb_b2), row(eb_g2), row(eb_be2), cb_t)
    idx_a = ia2.reshape(b)
    idx_b = ib2.reshape(b)

    gather = _make_sc_gather(cd, b)
    q_raw_a = gather(codebook, idx_a)
    q_raw_b = gather(codebook, idx_b)

    dec_a = _make_dec(b, bt, dim_a, hid, cd)
    dec_b = _make_dec(b, bt, dim_b, hid, cd)
    q_a, recon_a, ca = dec_a(z_a, q_raw_a,
                             ea_dw1.astype(_BF16), row(ea_db1),
                             ea_dw2.astype(_BF16), row(ea_db2))
    q_b, recon_b, cb = dec_b(z_b, q_raw_b,
                             eb_dw1.astype(_BF16), row(eb_db1),
                             eb_dw2.astype(_BF16), row(eb_db2))

    return (z_a, z_b, q_a, q_b, idx_a, idx_b, recon_a, recon_b,
            ca.reshape(()), cb.reshape(()))


# speedup vs baseline: 1.6609x; 1.0177x over previous
import functools

import jax
import jax.numpy as jnp
from jax import lax
from jax.experimental import pallas as pl
from jax.experimental.pallas import tpu as pltpu
from jax.experimental.pallas import tpu_sc as plsc

_F32 = jnp.float32
_BF16 = jnp.bfloat16


def _ln(x, g, b, eps=1e-5):
    m = jnp.mean(x, axis=-1, keepdims=True)
    v = jnp.mean((x - m) ** 2, axis=-1, keepdims=True)
    return (x - m) / jnp.sqrt(v + eps) * g + b


def _vq_search(z, cb_ref, e2_ref, it_ref, bt, num_codes, kt):
    hz2 = 0.5 * jnp.sum(z * z, axis=1, keepdims=True)
    cur_idx = None
    for t in range(num_codes // kt):
        cbt = cb_ref[:, pl.ds(t * kt, kt)]
        ze = lax.dot_general(z, cbt, (((1,), (0,)), ((), ())),
                             preferred_element_type=_F32)
        d = hz2 - ze + e2_ref[:, pl.ds(t * kt, kt)]
        m = jnp.min(d, axis=1, keepdims=True)
        am = jnp.min(jnp.where(d == m, it_ref[:, pl.ds(t * kt, kt)],
                               float(num_codes)),
                     axis=1, keepdims=True)
        if cur_idx is None:
            cur_min, cur_idx = m, am
        else:
            upd = m < cur_min
            cur_idx = jnp.where(upd, am, cur_idx)
            cur_min = jnp.where(upd, m, cur_min)
    return cur_idx.astype(jnp.int32)


def _enc_vq_body(bt, num_codes, kt,
                 x_ref, w1, b1, g1, be1, w2, b2, g2, be2, cb_ref,
                 z_ref, i_ref, e2_ref, it_ref):
    i = pl.program_id(0)

    @pl.when(i == 0)
    def _():
        cb = cb_ref[...]
        ones = jnp.ones((1, cb.shape[0]), dtype=_F32)
        e2_ref[...] = 0.5 * lax.dot_general(
            ones, cb * cb, (((1,), (0,)), ((), ())),
            preferred_element_type=_F32)
        it_ref[...] = lax.broadcasted_iota(
            jnp.int32, (1, num_codes), 1).astype(_F32)

    h = jax.nn.relu(_ln(jnp.dot(x_ref[...], w1[...],
                                preferred_element_type=_F32) + b1[...],
                        g1[...], be1[...]))
    z = _ln(jnp.dot(h, w2[...], preferred_element_type=_F32) + b2[...],
            g2[...], be2[...])
    z_ref[...] = z
    i_ref[...] = _vq_search(z, cb_ref, e2_ref, it_ref, bt, num_codes, kt)


def _dec_body(nsteps, inv_n,
              z_ref, qr_ref, dw1, db1, dw2, db2,
              qs_ref, r_ref, c_ref):
    i = pl.program_id(0)
    z = z_ref[...]
    qr = qr_ref[...]
    qs = z + (qr - z)
    qs_ref[...] = qs
    h = jax.nn.relu(jnp.dot(qs.astype(_BF16), dw1[...],
                            preferred_element_type=_F32) + db1[...])
    r_ref[...] = jnp.dot(h.astype(_BF16), dw2[...],
                         preferred_element_type=_F32) + db2[...]

    dz = z - qr
    p = jnp.sum(dz * dz).reshape(1, 1)

    @pl.when(i == 0)
    def _():
        c_ref[...] = jnp.zeros((1, 1), _F32)

    c_ref[...] += p

    @pl.when(i == nsteps - 1)
    def _():
        c_ref[...] = c_ref[...] * inv_n


def _make_sc_gather(cd, b):
    nc, ns = 2, 16
    nw = nc * ns
    bpw = b // nw
    ch = 128 if bpw % 128 == 0 else bpw
    nch = bpw // ch
    assert nch == 2, "pipelined gather below is written for 2 chunks/worker"
    mesh = plsc.VectorSubcoreMesh(core_axis_name="c", subcore_axis_name="s")

    @functools.partial(
        pl.kernel, mesh=mesh,
        out_type=jax.ShapeDtypeStruct((b, cd), _F32),
        scratch_types=[pltpu.VMEM((ch,), jnp.int32),
                       pltpu.VMEM((ch,), jnp.int32),
                       pltpu.VMEM((ch, cd), _F32),
                       pltpu.VMEM((ch, cd), _F32),
                       pltpu.SemaphoreType.DMA,
                       pltpu.SemaphoreType.DMA,
                       pltpu.SemaphoreType.DMA,
                       pltpu.SemaphoreType.DMA])
    def gather(cb_hbm, idx_hbm, q_hbm,
               i0, i1, r0, r1, sg0, sg1, ss0, ss1):
        wid = lax.axis_index("s") * nc + lax.axis_index("c")
        base = wid * bpw
        pltpu.sync_copy(idx_hbm.at[pl.ds(base, ch)], i0)
        pltpu.sync_copy(idx_hbm.at[pl.ds(base + ch, ch)], i1)
        g0 = pltpu.async_copy(cb_hbm.at[i0], r0, sg0)
        g1 = pltpu.async_copy(cb_hbm.at[i1], r1, sg1)
        g0.wait()
        s0 = pltpu.async_copy(r0, q_hbm.at[pl.ds(base, ch)], ss0)
        g1.wait()
        s1 = pltpu.async_copy(r1, q_hbm.at[pl.ds(base + ch, ch)], ss1)
        s0.wait()
        s1.wait()

    return gather


def _make_enc(b, bt, dim, hid, cd, num_codes, kt):
    nsteps = b // bt
    full = lambda shape: pl.BlockSpec(shape, lambda i: (0,) * len(shape))
    batch = lambda shape: pl.BlockSpec(shape, lambda i: (i,) + (0,) * (len(shape) - 1))
    return pl.pallas_call(
        functools.partial(_enc_vq_body, bt, num_codes, kt),
        grid=(nsteps,),
        in_specs=[
            batch((bt, dim)),
            full((dim, hid)), full((1, hid)), full((1, hid)), full((1, hid)),
            full((hid, cd)), full((1, cd)), full((1, cd)), full((1, cd)),
            full((cd, num_codes)),
        ],
        out_specs=[batch((bt, cd)), batch((bt, 1))],
        out_shape=[jax.ShapeDtypeStruct((b, cd), _F32),
                   jax.ShapeDtypeStruct((b, 1), jnp.int32)],
        scratch_shapes=[pltpu.VMEM((1, num_codes), _F32),
                        pltpu.VMEM((1, num_codes), _F32)],
        compiler_params=pltpu.CompilerParams(
            dimension_semantics=("arbitrary",)),
    )


def _make_dec(b, bt, dim, hid, cd):
    nsteps = b // bt
    full = lambda shape: pl.BlockSpec(shape, lambda i: (0,) * len(shape))
    batch = lambda shape: pl.BlockSpec(shape, lambda i: (i,) + (0,) * (len(shape) - 1))
    return pl.pallas_call(
        functools.partial(_dec_body, nsteps, 1.0 / (b * cd)),
        grid=(nsteps,),
        in_specs=[
            batch((bt, cd)), batch((bt, cd)),
            full((cd, hid)), full((1, hid)), full((hid, dim)), full((1, dim)),
        ],
        out_specs=[batch((bt, cd)), batch((bt, dim)), full((1, 1))],
        out_shape=[jax.ShapeDtypeStruct((b, cd), _F32),
                   jax.ShapeDtypeStruct((b, dim), _F32),
                   jax.ShapeDtypeStruct((1, 1), _F32)],
        compiler_params=pltpu.CompilerParams(
            dimension_semantics=("arbitrary",)),
    )


def kernel(x_a, x_b,
           ea_w1, ea_b1, ea_g1, ea_be1, ea_w2, ea_b2, ea_g2, ea_be2,
           ea_dw1, ea_db1, ea_dw2, ea_db2,
           eb_w1, eb_b1, eb_g1, eb_be1, eb_w2, eb_b2, eb_g2, eb_be2,
           eb_dw1, eb_db1, eb_dw2, eb_db2,
           codebook):
    b, dim_a = x_a.shape
    dim_b = x_b.shape[1]
    hid = ea_w1.shape[1]
    cd = ea_w2.shape[1]
    num_codes = codebook.shape[0]
    bt = 2048 if b % 2048 == 0 else b
    kt = 1024 if num_codes % 1024 == 0 else num_codes

    row = lambda v: v.reshape(1, -1)

    cb_t = codebook.T
    enc_a = _make_enc(b, bt, dim_a, hid, cd, num_codes, kt)
    enc_b = _make_enc(b, bt, dim_b, hid, cd, num_codes, kt)
    z_a, ia2 = enc_a(x_a, ea_w1, row(ea_b1), row(ea_g1), row(ea_be1),
                     ea_w2, row(ea_b2), row(ea_g2), row(ea_be2), cb_t)
    z_b, ib2 = enc_b(x_b, eb_w1, row(eb_b1), row(eb_g1), row(eb_be1),
                     eb_w2, row(eb_b2), row(eb_g2), row(eb_be2), cb_t)
    idx_a = ia2.reshape(b)
    idx_b = ib2.reshape(b)

    gather = _make_sc_gather(cd, b)
    q_raw_a = gather(codebook, idx_a)
    q_raw_b = gather(codebook, idx_b)

    dec_a = _make_dec(b, bt, dim_a, hid, cd)
    dec_b = _make_dec(b, bt, dim_b, hid, cd)
    q_a, recon_a, ca = dec_a(z_a, q_raw_a,
                             ea_dw1.astype(_BF16), row(ea_db1),
                             ea_dw2.astype(_BF16), row(ea_db2))
    q_b, recon_b, cb = dec_b(z_b, q_raw_b,
                             eb_dw1.astype(_BF16), row(eb_db1),
                             eb_dw2.astype(_BF16), row(eb_db2))

    return (z_a, z_b, q_a, q_b, idx_a, idx_b, recon_a, recon_b,
            ca.reshape(()), cb.reshape(()))


# speedup vs baseline: 1.7417x; 1.0486x over previous
import functools

import jax
import jax.numpy as jnp
from jax import lax
from jax.experimental import pallas as pl
from jax.experimental.pallas import tpu as pltpu
from jax.experimental.pallas import tpu_sc as plsc

_F32 = jnp.float32
_BF16 = jnp.bfloat16


def _ln(x, g, b, eps=1e-5):
    m = jnp.mean(x, axis=-1, keepdims=True)
    v = jnp.mean((x - m) ** 2, axis=-1, keepdims=True)
    return (x - m) / jnp.sqrt(v + eps) * g + b


def _vq_search(z, cb_ref, e2_ref, it_ref, bt, num_codes, kt):
    hz2 = 0.5 * jnp.sum(z * z, axis=1, keepdims=True)
    cur_idx = None
    for t in range(num_codes // kt):
        cbt = cb_ref[:, pl.ds(t * kt, kt)]
        ze = lax.dot_general(z, cbt, (((1,), (0,)), ((), ())),
                             preferred_element_type=_F32)
        d = hz2 - ze + e2_ref[:, pl.ds(t * kt, kt)]
        m = jnp.min(d, axis=1, keepdims=True)
        am = jnp.min(jnp.where(d == m, it_ref[:, pl.ds(t * kt, kt)],
                               float(num_codes)),
                     axis=1, keepdims=True)
        if cur_idx is None:
            cur_min, cur_idx = m, am
        else:
            upd = m < cur_min
            cur_idx = jnp.where(upd, am, cur_idx)
            cur_min = jnp.where(upd, m, cur_min)
    return cur_idx.astype(jnp.int32)


def _enc_vq_body(bt, num_codes, kt,
                 x_ref, w1, b1, g1, be1, w2, b2, g2, be2, cb_ref,
                 z_ref, i_ref, e2_ref, it_ref):
    i = pl.program_id(0)

    @pl.when(i == 0)
    def _():
        cb = cb_ref[...]
        ones = jnp.ones((1, cb.shape[0]), dtype=_F32)
        e2_ref[...] = 0.5 * lax.dot_general(
            ones, cb * cb, (((1,), (0,)), ((), ())),
            preferred_element_type=_F32)
        it_ref[...] = lax.broadcasted_iota(
            jnp.int32, (1, num_codes), 1).astype(_F32)

    h = jax.nn.relu(_ln(jnp.dot(x_ref[...], w1[...],
                                preferred_element_type=_F32) + b1[...],
                        g1[...], be1[...]))
    z = _ln(jnp.dot(h, w2[...], preferred_element_type=_F32) + b2[...],
            g2[...], be2[...])
    z_ref[...] = z
    i_ref[...] = _vq_search(z, cb_ref, e2_ref, it_ref, bt, num_codes, kt)


def _dec_body(nsteps, inv_n,
              z_ref, qr_ref, dw1, db1, dw2, db2,
              qs_ref, r_ref, c_ref):
    i = pl.program_id(0)
    z = z_ref[...]
    qr = qr_ref[...]
    qs = z + (qr - z)
    qs_ref[...] = qs
    h = jax.nn.relu(jnp.dot(qs.astype(_BF16), dw1[...],
                            preferred_element_type=_F32) + db1[...])
    r_ref[...] = jnp.dot(h.astype(_BF16), dw2[...],
                         preferred_element_type=_F32) + db2[...]

    dz = z - qr
    p = jnp.sum(dz * dz).reshape(1, 1)

    @pl.when(i == 0)
    def _():
        c_ref[...] = jnp.zeros((1, 1), _F32)

    c_ref[...] += p

    @pl.when(i == nsteps - 1)
    def _():
        c_ref[...] = c_ref[...] * inv_n


def _make_sc_gather(cd, b):
    nc, ns = 2, 16
    nw = nc * ns
    bpw = b // nw
    ch = 128 if bpw % 128 == 0 else bpw
    nch = bpw // ch
    assert nch == 2, "pipelined gather below is written for 2 chunks/worker"
    mesh = plsc.VectorSubcoreMesh(core_axis_name="c", subcore_axis_name="s")

    @functools.partial(
        pl.kernel, mesh=mesh,
        out_type=jax.ShapeDtypeStruct((b, cd), _F32),
        scratch_types=[pltpu.VMEM((ch,), jnp.int32),
                       pltpu.VMEM((ch,), jnp.int32),
                       pltpu.VMEM((ch, cd), _F32),
                       pltpu.VMEM((ch, cd), _F32),
                       pltpu.SemaphoreType.DMA,
                       pltpu.SemaphoreType.DMA,
                       pltpu.SemaphoreType.DMA,
                       pltpu.SemaphoreType.DMA])
    def gather(cb_hbm, idx_hbm, q_hbm,
               i0, i1, r0, r1, sg0, sg1, ss0, ss1):
        wid = lax.axis_index("s") * nc + lax.axis_index("c")
        base = wid * bpw
        pltpu.sync_copy(idx_hbm.at[pl.ds(base, ch)], i0)
        pltpu.sync_copy(idx_hbm.at[pl.ds(base + ch, ch)], i1)
        g0 = pltpu.async_copy(cb_hbm.at[i0], r0, sg0)
        g1 = pltpu.async_copy(cb_hbm.at[i1], r1, sg1)
        g0.wait()
        s0 = pltpu.async_copy(r0, q_hbm.at[pl.ds(base, ch)], ss0)
        g1.wait()
        s1 = pltpu.async_copy(r1, q_hbm.at[pl.ds(base + ch, ch)], ss1)
        s0.wait()
        s1.wait()

    return gather


def _make_enc(b, bt, dim, hid, cd, num_codes, kt):
    nsteps = b // bt
    full = lambda shape: pl.BlockSpec(shape, lambda i: (0,) * len(shape))
    batch = lambda shape: pl.BlockSpec(shape, lambda i: (i,) + (0,) * (len(shape) - 1))
    return pl.pallas_call(
        functools.partial(_enc_vq_body, bt, num_codes, kt),
        grid=(nsteps,),
        in_specs=[
            batch((bt, dim)),
            full((dim, hid)), full((1, hid)), full((1, hid)), full((1, hid)),
            full((hid, cd)), full((1, cd)), full((1, cd)), full((1, cd)),
            full((cd, num_codes)),
        ],
        out_specs=[batch((bt, cd)), batch((bt, 1))],
        out_shape=[jax.ShapeDtypeStruct((b, cd), _F32),
                   jax.ShapeDtypeStruct((b, 1), jnp.int32)],
        scratch_shapes=[pltpu.VMEM((1, num_codes), _F32),
                        pltpu.VMEM((1, num_codes), _F32)],
        compiler_params=pltpu.CompilerParams(
            dimension_semantics=("arbitrary",)),
    )


def _make_dec(b, bt, dim, hid, cd):
    nsteps = b // bt
    full = lambda shape: pl.BlockSpec(shape, lambda i: (0,) * len(shape))
    batch = lambda shape: pl.BlockSpec(shape, lambda i: (i,) + (0,) * (len(shape) - 1))
    return pl.pallas_call(
        functools.partial(_dec_body, nsteps, 1.0 / (b * cd)),
        grid=(nsteps,),
        in_specs=[
            batch((bt, cd)), batch((bt, cd)),
            full((cd, hid)), full((1, hid)), full((hid, dim)), full((1, dim)),
        ],
        out_specs=[batch((bt, cd)), batch((bt, dim)), full((1, 1))],
        out_shape=[jax.ShapeDtypeStruct((b, cd), _F32),
                   jax.ShapeDtypeStruct((b, dim), _F32),
                   jax.ShapeDtypeStruct((1, 1), _F32)],
        compiler_params=pltpu.CompilerParams(
            dimension_semantics=("arbitrary",)),
    )


def kernel(x_a, x_b,
           ea_w1, ea_b1, ea_g1, ea_be1, ea_w2, ea_b2, ea_g2, ea_be2,
           ea_dw1, ea_db1, ea_dw2, ea_db2,
           eb_w1, eb_b1, eb_g1, eb_be1, eb_w2, eb_b2, eb_g2, eb_be2,
           eb_dw1, eb_db1, eb_dw2, eb_db2,
           codebook):
    b, dim_a = x_a.shape
    dim_b = x_b.shape[1]
    hid = ea_w1.shape[1]
    cd = ea_w2.shape[1]
    num_codes = codebook.shape[0]
    bt = 1024 if b % 1024 == 0 else b
    kt = 2048 if num_codes % 2048 == 0 else num_codes

    row = lambda v: v.reshape(1, -1)

    cb_t = codebook.T
    enc_a = _make_enc(b, bt, dim_a, hid, cd, num_codes, kt)
    enc_b = _make_enc(b, bt, dim_b, hid, cd, num_codes, kt)
    z_a, ia2 = enc_a(x_a, ea_w1, row(ea_b1), row(ea_g1), row(ea_be1),
                     ea_w2, row(ea_b2), row(ea_g2), row(ea_be2), cb_t)
    z_b, ib2 = enc_b(x_b, eb_w1, row(eb_b1), row(eb_g1), row(eb_be1),
                     eb_w2, row(eb_b2), row(eb_g2), row(eb_be2), cb_t)
    idx_a = ia2.reshape(b)
    idx_b = ib2.reshape(b)

    gather = _make_sc_gather(cd, b)
    q_raw_a = gather(codebook, idx_a)
    q_raw_b = gather(codebook, idx_b)

    dec_a = _make_dec(b, bt, dim_a, hid, cd)
    dec_b = _make_dec(b, bt, dim_b, hid, cd)
    q_a, recon_a, ca = dec_a(z_a, q_raw_a,
                             ea_dw1.astype(_BF16), row(ea_db1),
                             ea_dw2.astype(_BF16), row(ea_db2))
    q_b, recon_b, cb = dec_b(z_b, q_raw_b,
                             eb_dw1.astype(_BF16), row(eb_db1),
                             eb_dw2.astype(_BF16), row(eb_db2))

    return (z_a, z_b, q_a, q_b, idx_a, idx_b, recon_a, recon_b,
            ca.reshape(()), cb.reshape(()))


# speedup vs baseline: 1.7685x; 1.0154x over previous
import functools

import jax
import jax.numpy as jnp
from jax import lax
from jax.experimental import pallas as pl
from jax.experimental.pallas import tpu as pltpu
from jax.experimental.pallas import tpu_sc as plsc

_F32 = jnp.float32
_BF16 = jnp.bfloat16


def _ln(x, g, b, eps=1e-5):
    m = jnp.mean(x, axis=-1, keepdims=True)
    v = jnp.mean((x - m) ** 2, axis=-1, keepdims=True)
    return (x - m) / jnp.sqrt(v + eps) * g + b


def _vq_search(z, cb_ref, e2_ref, it_ref, bt, num_codes, kt):
    hz2 = 0.5 * jnp.sum(z * z, axis=1, keepdims=True)
    cur_idx = None
    for t in range(num_codes // kt):
        cbt = cb_ref[:, pl.ds(t * kt, kt)]
        ze = lax.dot_general(z, cbt, (((1,), (0,)), ((), ())),
                             preferred_element_type=_F32)
        d = hz2 - ze + e2_ref[:, pl.ds(t * kt, kt)]
        m = jnp.min(d, axis=1, keepdims=True)
        am = jnp.min(jnp.where(d == m, it_ref[:, pl.ds(t * kt, kt)],
                               float(num_codes)),
                     axis=1, keepdims=True)
        if cur_idx is None:
            cur_min, cur_idx = m, am
        else:
            upd = m < cur_min
            cur_idx = jnp.where(upd, am, cur_idx)
            cur_min = jnp.where(upd, m, cur_min)
    return cur_idx.astype(jnp.int32)


def _enc_vq_body(bt, num_codes, kt,
                 x_ref, w1, b1, g1, be1, w2, b2, g2, be2, cb_ref,
                 z_ref, i_ref, e2_ref, it_ref):
    i = pl.program_id(0)

    @pl.when(i == 0)
    def _():
        cb = cb_ref[...]
        ones = jnp.ones((1, cb.shape[0]), dtype=_F32)
        e2_ref[...] = 0.5 * lax.dot_general(
            ones, cb * cb, (((1,), (0,)), ((), ())),
            preferred_element_type=_F32)
        it_ref[...] = lax.broadcasted_iota(
            jnp.int32, (1, num_codes), 1).astype(_F32)

    h = jax.nn.relu(_ln(jnp.dot(x_ref[...], w1[...],
                                preferred_element_type=_F32) + b1[...],
                        g1[...], be1[...]))
    z = _ln(jnp.dot(h, w2[...], preferred_element_type=_F32) + b2[...],
            g2[...], be2[...])
    z_ref[...] = z
    i_ref[...] = _vq_search(z, cb_ref, e2_ref, it_ref, bt, num_codes, kt)


def _dec_body(nsteps, inv_n,
              z_ref, qr_ref, dw1, db1, dw2, db2,
              qs_ref, r_ref, c_ref):
    i = pl.program_id(0)
    z = z_ref[...]
    qr = qr_ref[...]
    qs = z + (qr - z)
    qs_ref[...] = qs
    h = jax.nn.relu(jnp.dot(qs.astype(_BF16), dw1[...],
                            preferred_element_type=_F32) + db1[...])
    r_ref[...] = jnp.dot(h.astype(_BF16), dw2[...],
                         preferred_element_type=_F32) + db2[...]

    dz = z - qr
    p = jnp.sum(dz * dz).reshape(1, 1)

    @pl.when(i == 0)
    def _():
        c_ref[...] = jnp.zeros((1, 1), _F32)

    c_ref[...] += p

    @pl.when(i == nsteps - 1)
    def _():
        c_ref[...] = c_ref[...] * inv_n


def _make_sc_gather(cd, b):
    nc, ns = 2, 16
    nw = nc * ns
    bpw = b // nw
    ch = 128 if bpw % 128 == 0 else bpw
    nch = bpw // ch
    assert nch == 2, "pipelined gather below is written for 2 chunks/worker"
    mesh = plsc.VectorSubcoreMesh(core_axis_name="c", subcore_axis_name="s")

    @functools.partial(
        pl.kernel, mesh=mesh,
        out_type=jax.ShapeDtypeStruct((b, cd), _F32),
        scratch_types=[pltpu.VMEM((ch,), jnp.int32),
                       pltpu.VMEM((ch,), jnp.int32),
                       pltpu.VMEM((ch, cd), _F32),
                       pltpu.VMEM((ch, cd), _F32),
                       pltpu.SemaphoreType.DMA,
                       pltpu.SemaphoreType.DMA,
                       pltpu.SemaphoreType.DMA,
                       pltpu.SemaphoreType.DMA])
    def gather(cb_hbm, idx_hbm, q_hbm,
               i0, i1, r0, r1, sg0, sg1, ss0, ss1):
        wid = lax.axis_index("s") * nc + lax.axis_index("c")
        base = wid * bpw
        pltpu.sync_copy(idx_hbm.at[pl.ds(base, ch)], i0)
        pltpu.sync_copy(idx_hbm.at[pl.ds(base + ch, ch)], i1)
        g0 = pltpu.async_copy(cb_hbm.at[i0], r0, sg0)
        g1 = pltpu.async_copy(cb_hbm.at[i1], r1, sg1)
        g0.wait()
        s0 = pltpu.async_copy(r0, q_hbm.at[pl.ds(base, ch)], ss0)
        g1.wait()
        s1 = pltpu.async_copy(r1, q_hbm.at[pl.ds(base + ch, ch)], ss1)
        s0.wait()
        s1.wait()

    return gather


def _make_enc(b, bt, dim, hid, cd, num_codes, kt):
    nsteps = b // bt
    full = lambda shape: pl.BlockSpec(shape, lambda i: (0,) * len(shape))
    batch = lambda shape: pl.BlockSpec(shape, lambda i: (i,) + (0,) * (len(shape) - 1))
    return pl.pallas_call(
        functools.partial(_enc_vq_body, bt, num_codes, kt),
        grid=(nsteps,),
        in_specs=[
            batch((bt, dim)),
            full((dim, hid)), full((1, hid)), full((1, hid)), full((1, hid)),
            full((hid, cd)), full((1, cd)), full((1, cd)), full((1, cd)),
            full((cd, num_codes)),
        ],
        out_specs=[batch((bt, cd)), batch((bt, 1))],
        out_shape=[jax.ShapeDtypeStruct((b, cd), _F32),
                   jax.ShapeDtypeStruct((b, 1), jnp.int32)],
        scratch_shapes=[pltpu.VMEM((1, num_codes), _F32),
                        pltpu.VMEM((1, num_codes), _F32)],
        compiler_params=pltpu.CompilerParams(
            dimension_semantics=("arbitrary",)),
    )


def _make_dec(b, bt, dim, hid, cd):
    nsteps = b // bt
    full = lambda shape: pl.BlockSpec(shape, lambda i: (0,) * len(shape))
    batch = lambda shape: pl.BlockSpec(shape, lambda i: (i,) + (0,) * (len(shape) - 1))
    return pl.pallas_call(
        functools.partial(_dec_body, nsteps, 1.0 / (b * cd)),
        grid=(nsteps,),
        in_specs=[
            batch((bt, cd)), batch((bt, cd)),
            full((cd, hid)), full((1, hid)), full((hid, dim)), full((1, dim)),
        ],
        out_specs=[batch((bt, cd)), batch((bt, dim)), full((1, 1))],
        out_shape=[jax.ShapeDtypeStruct((b, cd), _F32),
                   jax.ShapeDtypeStruct((b, dim), _F32),
                   jax.ShapeDtypeStruct((1, 1), _F32)],
        compiler_params=pltpu.CompilerParams(
            dimension_semantics=("arbitrary",)),
    )


def kernel(x_a, x_b,
           ea_w1, ea_b1, ea_g1, ea_be1, ea_w2, ea_b2, ea_g2, ea_be2,
           ea_dw1, ea_db1, ea_dw2, ea_db2,
           eb_w1, eb_b1, eb_g1, eb_be1, eb_w2, eb_b2, eb_g2, eb_be2,
           eb_dw1, eb_db1, eb_dw2, eb_db2,
           codebook):
    b, dim_a = x_a.shape
    dim_b = x_b.shape[1]
    hid = ea_w1.shape[1]
    cd = ea_w2.shape[1]
    num_codes = codebook.shape[0]
    bt = 1024 if b % 1024 == 0 else b
    kt = 1024 if num_codes % 1024 == 0 else num_codes

    row = lambda v: v.reshape(1, -1)

    cb_t = codebook.T
    enc_a = _make_enc(b, bt, dim_a, hid, cd, num_codes, kt)
    enc_b = _make_enc(b, bt, dim_b, hid, cd, num_codes, kt)
    z_a, ia2 = enc_a(x_a, ea_w1, row(ea_b1), row(ea_g1), row(ea_be1),
                     ea_w2, row(ea_b2), row(ea_g2), row(ea_be2), cb_t)
    z_b, ib2 = enc_b(x_b, eb_w1, row(eb_b1), row(eb_g1), row(eb_be1),
                     eb_w2, row(eb_b2), row(eb_g2), row(eb_be2), cb_t)
    idx_a = ia2.reshape(b)
    idx_b = ib2.reshape(b)

    gather = _make_sc_gather(cd, b)
    q_raw_a = gather(codebook, idx_a)
    q_raw_b = gather(codebook, idx_b)

    dec_a = _make_dec(b, bt, dim_a, hid, cd)
    dec_b = _make_dec(b, bt, dim_b, hid, cd)
    q_a, recon_a, ca = dec_a(z_a, q_raw_a,
                             ea_dw1.astype(_BF16), row(ea_db1),
                             ea_dw2.astype(_BF16), row(ea_db2))
    q_b, recon_b, cb = dec_b(z_b, q_raw_b,
                             eb_dw1.astype(_BF16), row(eb_db1),
                             eb_dw2.astype(_BF16), row(eb_db2))

    return (z_a, z_b, q_a, q_b, idx_a, idx_b, recon_a, recon_b,
            ca.reshape(()), cb.reshape(()))
